# Initial kernel scaffold; baseline (speedup 1.0000x reference)
import functools

import jax
import jax.numpy as jnp
from jax import lax
from jax.experimental import pallas as pl
from jax.experimental.pallas import tpu as pltpu
from jax.experimental.pallas import tpu_sc as plsc

N = 10000
E = 320000
D = 128

NC = 2
NS = 16
NW = NC * NS

NPAD = 10240
STRIPE = NPAD // NS

EPT = E // NW
KA = 128
NKA = EPT // KA
TA = EPT - NKA * KA

KS = 64
NKS = EPT // KS
TS = EPT - NKS * KS

_mesh = plsc.VectorSubcoreMesh(
    core_axis_name="c", subcore_axis_name="s", num_cores=NC, num_subcores=NS
)


@functools.partial(
    pl.kernel,
    out_type=(
        jax.ShapeDtypeStruct((NC, NPAD, D), jnp.float32),
        jax.ShapeDtypeStruct((NC, NPAD), jnp.float32),
    ),
    mesh=_mesh,
    scratch_types=[
        pltpu.VMEM_SHARED((NPAD, D), jnp.float32),
        pltpu.VMEM_SHARED((NPAD,), jnp.float32),
        pltpu.VMEM((KA, D), jnp.float32),
        pltpu.VMEM((KA,), jnp.int32),
        pltpu.VMEM((KA,), jnp.int32),
        pltpu.VMEM((KA,), jnp.float32),
        pltpu.VMEM((TA, D), jnp.float32),
        pltpu.VMEM((TA,), jnp.int32),
        pltpu.VMEM((TA,), jnp.int32),
        pltpu.VMEM((TA,), jnp.float32),
        pltpu.SemaphoreType.DMA,
    ],
)
def _sc_aggregate(feat, srcl, dstl, z2d, z1d, out_agg, out_deg,
                  agg_sh, deg_sh, rows_v, isv, idv, ones_v,
                  rows_t, isv_t, idv_t, ones_t, sem):
    c = lax.axis_index("c")
    s = lax.axis_index("s")
    wid = c * NS + s

    pltpu.sync_copy(z2d.at[pl.ds(s * STRIPE, STRIPE)],
                    agg_sh.at[pl.ds(s * STRIPE, STRIPE)])

    @pl.when(s == 0)
    def _():
        pltpu.sync_copy(z1d, deg_sh)

    for i in range(KA // 16):
        ones_v[pl.ds(i * 16, 16)] = jnp.full((16,), 1.0, jnp.float32)
    ones_t[...] = jnp.full((TA,), 1.0, jnp.float32)

    plsc.subcore_barrier()

    base = wid * EPT

    def do_chunk(off, rows, i_s, i_d, ones):
        pltpu.sync_copy(srcl.at[pl.ds(off, i_s.shape[0])], i_s)
        pltpu.sync_copy(dstl.at[pl.ds(off, i_d.shape[0])], i_d)
        pltpu.async_copy(feat.at[i_s], rows, sem).wait()
        pltpu.sync_copy(rows, agg_sh.at[i_d], add=True)
        pltpu.sync_copy(ones, deg_sh.at[i_d], add=True)

    def loop_body(i, carry):
        off = pl.multiple_of(base + i * KA, 8)
        do_chunk(off, rows_v, isv, idv, ones_v)
        return carry

    lax.fori_loop(0, NKA, loop_body, 0)
    do_chunk(pl.multiple_of(base + NKA * KA, 8), rows_t, isv_t, idv_t, ones_t)

    plsc.subcore_barrier()

    pltpu.sync_copy(agg_sh.at[pl.ds(s * STRIPE, STRIPE)],
                    out_agg.at[c, pl.ds(s * STRIPE, STRIPE)])
    pltpu.sync_copy(deg_sh.at[pl.ds(s * STRIPE, STRIPE)],
                    out_deg.at[c, pl.ds(s * STRIPE, STRIPE)])


@functools.partial(
    pl.kernel,
    out_type=jax.ShapeDtypeStruct((16, E), jnp.float32),
    mesh=_mesh,
    scratch_types=[
        pltpu.VMEM((KS, D), jnp.float32),
        pltpu.VMEM((KS, D), jnp.float32),
        pltpu.VMEM((KS,), jnp.int32),
        pltpu.VMEM((KS,), jnp.int32),
        pltpu.VMEM((16, KS), jnp.float32),
        pltpu.VMEM((TS, D), jnp.float32),
        pltpu.VMEM((TS, D), jnp.float32),
        pltpu.VMEM((TS,), jnp.int32),
        pltpu.VMEM((TS,), jnp.int32),
        pltpu.VMEM((16, TS), jnp.float32),
        pltpu.SemaphoreType.DMA,
        pltpu.SemaphoreType.DMA,
    ],
)
def _sc_score(h, srcl, dstl, out_p,
              hs_v, hd_v, isv, idv, p_v,
              hs_t, hd_t, isv_t, idv_t, p_t, sem_a, sem_b):
    c = lax.axis_index("c")
    s = lax.axis_index("s")
    wid = c * NS + s
    base = wid * EPT
    lane = lax.iota(jnp.int32, 16)

    def do_chunk(off, hs, hd, i_s, i_d, p):
        k = i_s.shape[0]
        pltpu.sync_copy(srcl.at[pl.ds(off, k)], i_s)
        pltpu.sync_copy(dstl.at[pl.ds(off, k)], i_d)
        ca = pltpu.async_copy(h.at[i_s], hs, sem_a)
        cb = pltpu.async_copy(h.at[i_d], hd, sem_b)
        ca.wait()
        cb.wait()
        for e in range(k):
            acc = hs[e, pl.ds(0, 16)] * hd[e, pl.ds(0, 16)]
            for j in range(1, D // 16):
                acc = acc + hs[e, pl.ds(j * 16, 16)] * hd[e, pl.ds(j * 16, 16)]
            plsc.store_scatter(p, [lane, jnp.full((16,), e, jnp.int32)], acc)
        pltpu.sync_copy(p, out_p.at[:, pl.ds(off, k)])

    def loop_body(i, carry):
        off = pl.multiple_of(base + i * KS, 8)
        do_chunk(off, hs_v, hd_v, isv, idv, p_v)
        return carry

    lax.fori_loop(0, NKS, loop_body, 0)
    do_chunk(pl.multiple_of(base + NKS * KS, 8), hs_t, hd_t, isv_t, idv_t, p_t)


BN = 1024


def _dense_body(x_ref, a0_ref, a1_ref, df_ref, ws_ref, wn_ref, b_ref, o_ref):
    a = a0_ref[0] + a1_ref[0]
    hn = a / df_ref[...]
    h = jnp.dot(x_ref[...], ws_ref[...], preferred_element_type=jnp.float32)
    h = h + jnp.dot(hn, wn_ref[...], preferred_element_type=jnp.float32)
    o_ref[...] = h + b_ref[...]


def _dense(x_p, agg_p, deg_full, ws_t, wn_t, b):
    return pl.pallas_call(
        _dense_body,
        grid=(NPAD // BN,),
        in_specs=[
            pl.BlockSpec((BN, D), lambda i: (i, 0)),
            pl.BlockSpec((1, BN, D), lambda i: (0, i, 0)),
            pl.BlockSpec((1, BN, D), lambda i: (1, i, 0)),
            pl.BlockSpec((BN, D), lambda i: (i, 0)),
            pl.BlockSpec((D, D), lambda i: (0, 0)),
            pl.BlockSpec((D, D), lambda i: (0, 0)),
            pl.BlockSpec((1, D), lambda i: (0, 0)),
        ],
        out_specs=pl.BlockSpec((BN, D), lambda i: (i, 0)),
        out_shape=jax.ShapeDtypeStruct((NPAD, D), jnp.float32),
    )(x_p, agg_p, agg_p, deg_full, ws_t, wn_t, b)


EROWS = E // D
BER = 250


def _finish_body(p_ref, o_ref, mn_ref, mx_ref):
    ph = pl.program_id(0)
    i = pl.program_id(1)
    s = jnp.sum(p_ref[...], axis=0)

    @pl.when(ph == 0)
    def _():
        m = jnp.min(s)
        mm = jnp.max(s)

        @pl.when(i == 0)
        def _():
            mn_ref[0, 0] = m
            mx_ref[0, 0] = mm

        @pl.when(i > 0)
        def _():
            mn_ref[0, 0] = jnp.minimum(mn_ref[0, 0], m)
            mx_ref[0, 0] = jnp.maximum(mx_ref[0, 0], mm)

    @pl.when(ph == 1)
    def _():
        o_ref[...] = (s - mn_ref[0, 0]) / (mx_ref[0, 0] - mn_ref[0, 0])


def _finish(p3):
    return pl.pallas_call(
        _finish_body,
        grid=(2, EROWS // BER),
        in_specs=[pl.BlockSpec((16, BER, D), lambda p, i: (0, i, 0))],
        out_specs=pl.BlockSpec((BER, D), lambda p, i: (i, 0)),
        out_shape=jax.ShapeDtypeStruct((EROWS, D), jnp.float32),
        scratch_shapes=[
            pltpu.SMEM((1, 1), jnp.float32),
            pltpu.SMEM((1, 1), jnp.float32),
        ],
    )(p3)


@jax.jit
def kernel(x, edge_index, W1_self, W1_neigh, b1, W2_self, W2_neigh, b2):
    src = edge_index[0]
    dst = edge_index[1]

    x_p = jnp.pad(x, ((0, NPAD - N), (0, 0)))
    z2d = jnp.zeros((NPAD, D), jnp.float32)
    z1d = jnp.zeros((NPAD,), jnp.float32)

    agg1, deg = _sc_aggregate(x_p, src, dst, z2d, z1d)
    degsum = jnp.maximum(deg[0] + deg[1], 1.0)
    deg_full = jnp.broadcast_to(degsum[:, None], (NPAD, D))

    h1 = _dense(x_p, agg1, deg_full, W1_self.T, W1_neigh.T, b1[None, :])
    agg2, _ = _sc_aggregate(h1, src, dst, z2d, z1d)
    h2 = _dense(h1, agg2, deg_full, W2_self.T, W2_neigh.T, b2[None, :])

    p = _sc_score(h2, src, dst)
    p3 = p.reshape(16, EROWS, D)
    label = _finish(p3).reshape(E)
    return label


# speedup vs baseline: 3.5688x; 3.5688x over previous
import functools

import jax
import jax.numpy as jnp
from jax import lax
from jax.experimental import pallas as pl
from jax.experimental.pallas import tpu as pltpu
from jax.experimental.pallas import tpu_sc as plsc

N = 10000
E = 320000
D = 128

NC = 2
NS = 16
NW = NC * NS

NPAD = 10240
STRIPE = NPAD // NS

EPT = E // NW
KA = 128
NKA = EPT // KA
TA = EPT - NKA * KA

KS = 64
NKS = EPT // KS
TS = EPT - NKS * KS

_mesh = plsc.VectorSubcoreMesh(
    core_axis_name="c", subcore_axis_name="s", num_cores=NC, num_subcores=NS
)


@functools.partial(
    pl.kernel,
    out_type=(
        jax.ShapeDtypeStruct((NC, NPAD, D), jnp.float32),
        jax.ShapeDtypeStruct((NC, NPAD), jnp.float32),
    ),
    mesh=_mesh,
    scratch_types=[
        pltpu.VMEM_SHARED((NPAD, D), jnp.float32),
        pltpu.VMEM_SHARED((NPAD,), jnp.float32),
        pltpu.VMEM((KA, D), jnp.float32),
        pltpu.VMEM((KA,), jnp.int32),
        pltpu.VMEM((KA,), jnp.int32),
        pltpu.VMEM((KA,), jnp.float32),
        pltpu.VMEM((TA, D), jnp.float32),
        pltpu.VMEM((TA,), jnp.int32),
        pltpu.VMEM((TA,), jnp.int32),
        pltpu.VMEM((TA,), jnp.float32),
        pltpu.SemaphoreType.DMA,
    ],
)
def _sc_aggregate(feat, srcl, dstl, z2d, z1d, out_agg, out_deg,
                  agg_sh, deg_sh, rows_v, isv, idv, ones_v,
                  rows_t, isv_t, idv_t, ones_t, sem):
    c = lax.axis_index("c")
    s = lax.axis_index("s")
    wid = c * NS + s

    pltpu.sync_copy(z2d.at[pl.ds(s * STRIPE, STRIPE)],
                    agg_sh.at[pl.ds(s * STRIPE, STRIPE)])

    @pl.when(s == 0)
    def _():
        pltpu.sync_copy(z1d, deg_sh)

    for i in range(KA // 16):
        ones_v[pl.ds(i * 16, 16)] = jnp.full((16,), 1.0, jnp.float32)
    ones_t[...] = jnp.full((TA,), 1.0, jnp.float32)

    plsc.subcore_barrier()

    base = wid * EPT

    def do_chunk(off, rows, i_s, i_d, ones):
        pltpu.sync_copy(srcl.at[pl.ds(off, i_s.shape[0])], i_s)
        pltpu.sync_copy(dstl.at[pl.ds(off, i_d.shape[0])], i_d)
        pltpu.async_copy(feat.at[i_s], rows, sem).wait()
        pltpu.sync_copy(rows, agg_sh.at[i_d], add=True)
        pltpu.sync_copy(ones, deg_sh.at[i_d], add=True)

    def loop_body(i, carry):
        off = pl.multiple_of(base + i * KA, 8)
        do_chunk(off, rows_v, isv, idv, ones_v)
        return carry

    lax.fori_loop(0, NKA, loop_body, 0)
    do_chunk(pl.multiple_of(base + NKA * KA, 8), rows_t, isv_t, idv_t, ones_t)

    plsc.subcore_barrier()

    pltpu.sync_copy(agg_sh.at[pl.ds(s * STRIPE, STRIPE)],
                    out_agg.at[c, pl.ds(s * STRIPE, STRIPE)])
    pltpu.sync_copy(deg_sh.at[pl.ds(s * STRIPE, STRIPE)],
                    out_deg.at[c, pl.ds(s * STRIPE, STRIPE)])


@functools.partial(
    pl.kernel,
    out_type=jax.ShapeDtypeStruct((E, 16), jnp.float32),
    mesh=_mesh,
    scratch_types=[
        pltpu.VMEM((KS, D), jnp.float32),
        pltpu.VMEM((KS, D), jnp.float32),
        pltpu.VMEM((KS,), jnp.int32),
        pltpu.VMEM((KS,), jnp.int32),
        pltpu.VMEM((KS, 16), jnp.float32),
        pltpu.VMEM((TS, D), jnp.float32),
        pltpu.VMEM((TS, D), jnp.float32),
        pltpu.VMEM((TS,), jnp.int32),
        pltpu.VMEM((TS,), jnp.int32),
        pltpu.VMEM((TS, 16), jnp.float32),
        pltpu.SemaphoreType.DMA,
        pltpu.SemaphoreType.DMA,
    ],
)
def _sc_score(h, srcl, dstl, out_p,
              hs_v, hd_v, isv, idv, p_v,
              hs_t, hd_t, isv_t, idv_t, p_t, sem_a, sem_b):
    c = lax.axis_index("c")
    s = lax.axis_index("s")
    wid = c * NS + s
    base = wid * EPT

    def do_chunk(off, hs, hd, i_s, i_d, p):
        k = i_s.shape[0]
        pltpu.sync_copy(srcl.at[pl.ds(off, k)], i_s)
        pltpu.sync_copy(dstl.at[pl.ds(off, k)], i_d)
        ca = pltpu.async_copy(h.at[i_s], hs, sem_a)
        cb = pltpu.async_copy(h.at[i_d], hd, sem_b)
        ca.wait()
        cb.wait()
        for e in range(k):
            acc = hs[e, pl.ds(0, 16)] * hd[e, pl.ds(0, 16)]
            for j in range(1, D // 16):
                acc = acc + hs[e, pl.ds(j * 16, 16)] * hd[e, pl.ds(j * 16, 16)]
            p[e, :] = acc
        pltpu.sync_copy(p, out_p.at[pl.ds(off, k)])

    def loop_body(i, carry):
        off = pl.multiple_of(base + i * KS, 8)
        do_chunk(off, hs_v, hd_v, isv, idv, p_v)
        return carry

    lax.fori_loop(0, NKS, loop_body, 0)
    do_chunk(pl.multiple_of(base + NKS * KS, 8), hs_t, hd_t, isv_t, idv_t, p_t)


BN = 1024


def _dense_body(x_ref, a0_ref, a1_ref, df_ref, ws_ref, wn_ref, b_ref, o_ref):
    a = a0_ref[0] + a1_ref[0]
    hn = a / df_ref[...]
    h = jnp.dot(x_ref[...], ws_ref[...], preferred_element_type=jnp.float32)
    h = h + jnp.dot(hn, wn_ref[...], preferred_element_type=jnp.float32)
    o_ref[...] = h + b_ref[...]


def _dense(x_p, agg_p, deg_full, ws_t, wn_t, b):
    return pl.pallas_call(
        _dense_body,
        grid=(NPAD // BN,),
        in_specs=[
            pl.BlockSpec((BN, D), lambda i: (i, 0)),
            pl.BlockSpec((1, BN, D), lambda i: (0, i, 0)),
            pl.BlockSpec((1, BN, D), lambda i: (1, i, 0)),
            pl.BlockSpec((BN, D), lambda i: (i, 0)),
            pl.BlockSpec((D, D), lambda i: (0, 0)),
            pl.BlockSpec((D, D), lambda i: (0, 0)),
            pl.BlockSpec((1, D), lambda i: (0, 0)),
        ],
        out_specs=pl.BlockSpec((BN, D), lambda i: (i, 0)),
        out_shape=jax.ShapeDtypeStruct((NPAD, D), jnp.float32),
    )(x_p, agg_p, agg_p, deg_full, ws_t, wn_t, b)


PROWS = E * 16 // D
BR = 4000


def _finish_body(p_ref, g_ref, o_ref, mn_ref, mx_ref):
    ph = pl.program_id(0)
    i = pl.program_id(1)
    s = jnp.dot(p_ref[...], g_ref[...], preferred_element_type=jnp.float32)

    @pl.when(ph == 0)
    def _():
        m = jnp.min(s)
        mm = jnp.max(s)

        @pl.when(i == 0)
        def _():
            mn_ref[0, 0] = m
            mx_ref[0, 0] = mm

        @pl.when(i > 0)
        def _():
            mn_ref[0, 0] = jnp.minimum(mn_ref[0, 0], m)
            mx_ref[0, 0] = jnp.maximum(mx_ref[0, 0], mm)

    @pl.when(ph == 1)
    def _():
        o_ref[...] = (s - mn_ref[0, 0]) / (mx_ref[0, 0] - mn_ref[0, 0])


def _finish(p2, g):
    return pl.pallas_call(
        _finish_body,
        grid=(2, PROWS // BR),
        in_specs=[
            pl.BlockSpec((BR, D), lambda p, i: (i, 0)),
            pl.BlockSpec((D, D), lambda p, i: (0, 0)),
        ],
        out_specs=pl.BlockSpec((BR, D), lambda p, i: (i, 0)),
        out_shape=jax.ShapeDtypeStruct((PROWS, D), jnp.float32),
        scratch_shapes=[
            pltpu.SMEM((1, 1), jnp.float32),
            pltpu.SMEM((1, 1), jnp.float32),
        ],
    )(p2, g)


@jax.jit
def kernel(x, edge_index, W1_self, W1_neigh, b1, W2_self, W2_neigh, b2):
    src = edge_index[0]
    dst = edge_index[1]

    x_p = jnp.pad(x, ((0, NPAD - N), (0, 0)))
    z2d = jnp.zeros((NPAD, D), jnp.float32)
    z1d = jnp.zeros((NPAD,), jnp.float32)

    agg1, deg = _sc_aggregate(x_p, src, dst, z2d, z1d)
    degsum = jnp.maximum(deg[0] + deg[1], 1.0)
    deg_full = jnp.broadcast_to(degsum[:, None], (NPAD, D))

    h1 = _dense(x_p, agg1, deg_full, W1_self.T, W1_neigh.T, b1[None, :])
    agg2, _ = _sc_aggregate(h1, src, dst, z2d, z1d)
    h2 = _dense(h1, agg2, deg_full, W2_self.T, W2_neigh.T, b2[None, :])

    p = _sc_score(h2, src, dst)
    p2 = p.reshape(PROWS, D)
    col = jnp.arange(D, dtype=jnp.int32)
    g = (col[:, None] // 16 == col[None, :] % 8).astype(jnp.float32)
    label = _finish(p2, g)[:, :8].reshape(E)
    return label


# speedup vs baseline: 5.3111x; 1.4882x over previous
import functools

import jax
import jax.numpy as jnp
from jax import lax
from jax.experimental import pallas as pl
from jax.experimental.pallas import tpu as pltpu
from jax.experimental.pallas import tpu_sc as plsc

N = 10000
E = 320000
D = 128

NC = 2
NS = 16
NW = NC * NS

NPAD = 10112
STRIPE = NPAD // NS

EPT = E // NW
KA = 64
NKA = EPT // KA
TA = EPT - NKA * KA

KS = KA
NKS = NKA
TS = TA

_mesh = plsc.VectorSubcoreMesh(
    core_axis_name="c", subcore_axis_name="s", num_cores=NC, num_subcores=NS
)


@functools.partial(
    pl.kernel,
    out_type=(
        jax.ShapeDtypeStruct((NC, NPAD, D), jnp.float32),
        jax.ShapeDtypeStruct((NC, NPAD), jnp.float32),
    ),
    mesh=_mesh,
    scratch_types=[
        pltpu.VMEM_SHARED((NPAD, D), jnp.float32),
        pltpu.VMEM_SHARED((NPAD,), jnp.float32),
        pltpu.VMEM((KA, D), jnp.float32),
        pltpu.VMEM((KA, D), jnp.float32),
        pltpu.VMEM((KA, D), jnp.float32),
        pltpu.VMEM((KA,), jnp.int32),
        pltpu.VMEM((KA,), jnp.int32),
        pltpu.VMEM((KA,), jnp.int32),
        pltpu.VMEM((KA,), jnp.int32),
        pltpu.VMEM((KA,), jnp.int32),
        pltpu.VMEM((KA,), jnp.int32),
        pltpu.VMEM((KA,), jnp.float32),
        pltpu.VMEM((TA, D), jnp.float32),
        pltpu.VMEM((TA,), jnp.int32),
        pltpu.VMEM((TA,), jnp.int32),
        pltpu.VMEM((TA,), jnp.float32),
        pltpu.SemaphoreType.DMA,
        pltpu.SemaphoreType.DMA,
        pltpu.SemaphoreType.DMA,
        pltpu.SemaphoreType.DMA,
        pltpu.SemaphoreType.DMA,
        pltpu.SemaphoreType.DMA,
        pltpu.SemaphoreType.DMA,
        pltpu.SemaphoreType.DMA,
        pltpu.SemaphoreType.DMA,
    ],
)
def _sc_aggregate(feat, srcl, dstl, z2d, z1d,
                  out_agg, out_deg,
                  agg_sh, deg_sh, rows0, rows1, rows2,
                  is0, is1, is2, id0, id1, id2, ones_v,
                  rows_t, isv_t, idv_t, ones_t,
                  si0, si1, si2, sg0, sg1, sg2, ss0, ss1, ss2):
    c = lax.axis_index("c")
    s = lax.axis_index("s")
    wid = c * NS + s
    rows = [rows0, rows1, rows2]
    isv = [is0, is1, is2]
    idv = [id0, id1, id2]
    sem_i = [si0, si1, si2]
    sem_g = [sg0, sg1, sg2]
    sem_s = [ss0, ss1, ss2]

    pltpu.sync_copy(z2d.at[pl.ds(s * STRIPE, STRIPE)],
                    agg_sh.at[pl.ds(s * STRIPE, STRIPE)])

    @pl.when(s == 0)
    def _():
        pltpu.sync_copy(z1d, deg_sh)

    for i in range(KA // 16):
        ones_v[pl.ds(i * 16, 16)] = jnp.full((16,), 1.0, jnp.float32)
    ones_t[...] = jnp.full((TA,), 1.0, jnp.float32)

    plsc.subcore_barrier()

    base = wid * EPT

    desc_i = [None] * NKA
    desc_g = [None] * NKA
    desc_s = [None] * NKA

    for t in range(NKA + 2):
        b = t % 3
        if t >= 3:
            desc_s[t - 3][0].wait()
            desc_s[t - 3][1].wait()
        if t < NKA:
            off = pl.multiple_of(base + t * KA, 8)
            d1 = pltpu.async_copy(srcl.at[pl.ds(off, KA)], isv[b], sem_i[b])
            d2 = pltpu.async_copy(dstl.at[pl.ds(off, KA)], idv[b], sem_i[b])
            desc_i[t] = (d1, d2)
        if 0 <= t - 1 < NKA:
            g = t - 1
            bg = g % 3
            desc_i[g][0].wait()
            desc_i[g][1].wait()
            desc_g[g] = pltpu.async_copy(feat.at[isv[bg]], rows[bg], sem_g[bg])
        if 0 <= t - 2 < NKA:
            sc = t - 2
            bs = sc % 3
            desc_g[sc].wait()
            d1 = pltpu.async_copy(rows[bs], agg_sh.at[idv[bs]], sem_s[bs],
                                  add=True)
            d2 = pltpu.async_copy(ones_v, deg_sh.at[idv[bs]], sem_s[bs],
                                  add=True)
            desc_s[sc] = (d1, d2)
    desc_s[NKA - 1][0].wait()
    desc_s[NKA - 1][1].wait()

    toff = pl.multiple_of(base + NKA * KA, 8)
    pltpu.sync_copy(srcl.at[pl.ds(toff, TA)], isv_t)
    pltpu.sync_copy(dstl.at[pl.ds(toff, TA)], idv_t)
    pltpu.async_copy(feat.at[isv_t], rows_t, sg0).wait()
    pltpu.sync_copy(rows_t, agg_sh.at[idv_t], add=True)
    pltpu.sync_copy(ones_t, deg_sh.at[idv_t], add=True)

    plsc.subcore_barrier()

    pltpu.sync_copy(agg_sh.at[pl.ds(s * STRIPE, STRIPE)],
                    out_agg.at[c, pl.ds(s * STRIPE, STRIPE)])

    @pl.when(s == 0)
    def _():
        pltpu.sync_copy(deg_sh, out_deg.at[c])


@functools.partial(
    pl.kernel,
    out_type=jax.ShapeDtypeStruct((E, 16), jnp.float32),
    mesh=_mesh,
    scratch_types=[
        pltpu.VMEM((NKS, KS), jnp.int32),
        pltpu.VMEM((NKS, KS), jnp.int32),
        pltpu.VMEM((KS, D), jnp.float32),
        pltpu.VMEM((KS, D), jnp.float32),
        pltpu.VMEM((KS, D), jnp.float32),
        pltpu.VMEM((KS, D), jnp.float32),
        pltpu.VMEM((KS, 16), jnp.float32),
        pltpu.VMEM((KS, 16), jnp.float32),
        pltpu.VMEM((TS, D), jnp.float32),
        pltpu.VMEM((TS, D), jnp.float32),
        pltpu.VMEM((TS,), jnp.int32),
        pltpu.VMEM((TS,), jnp.int32),
        pltpu.VMEM((TS, 16), jnp.float32),
        pltpu.SemaphoreType.DMA,
        pltpu.SemaphoreType.DMA,
        pltpu.SemaphoreType.DMA,
        pltpu.SemaphoreType.DMA,
        pltpu.SemaphoreType.DMA,
        pltpu.SemaphoreType.DMA,
    ],
)
def _sc_score(h, src_s, dst_s, src_st, dst_st, out_p,
              isv_all, idv_all, hs0, hs1, hd0, hd1, p0, p1,
              hs_t, hd_t, isv_t, idv_t, p_t,
              sa0, sa1, sb0, sb1, so0, so1):
    c = lax.axis_index("c")
    s = lax.axis_index("s")
    wid = c * NS + s
    base = wid * EPT
    hs = [hs0, hs1]
    hd = [hd0, hd1]
    p = [p0, p1]
    sem_a = [sa0, sa1]
    sem_b = [sb0, sb1]
    sem_o = [so0, so1]

    pltpu.sync_copy(src_s.at[wid], isv_all)
    pltpu.sync_copy(dst_s.at[wid], idv_all)
    pltpu.sync_copy(src_st.at[wid], isv_t)
    pltpu.sync_copy(dst_st.at[wid], idv_t)

    def compute(hsr, hdr, pr, k):
        for e in range(k):
            acc = hsr[e, pl.ds(0, 16)] * hdr[e, pl.ds(0, 16)]
            for j in range(1, D // 16):
                acc = acc + hsr[e, pl.ds(j * 16, 16)] * hdr[e, pl.ds(j * 16, 16)]
            pr[e, :] = acc

    pltpu.async_copy(h.at[isv_all.at[0]], hs[0], sem_a[0])
    pltpu.async_copy(h.at[idv_all.at[0]], hd[0], sem_b[0])
    pltpu.async_copy(h.at[isv_all.at[1]], hs[1], sem_a[1])
    pltpu.async_copy(h.at[idv_all.at[1]], hd[1], sem_b[1])

    def half(i, b):
        @pl.when(i >= 2)
        def _():
            pltpu.make_async_copy(
                p[b], out_p.at[pl.ds(0, KS)], sem_o[b]).wait()

        pltpu.make_async_copy(h.at[pl.ds(0, KS)], hs[b], sem_a[b]).wait()
        pltpu.make_async_copy(h.at[pl.ds(0, KS)], hd[b], sem_b[b]).wait()

        compute(hs[b], hd[b], p[b], KS)

        off = pl.multiple_of(base + i * KS, 8)
        pltpu.async_copy(p[b], out_p.at[pl.ds(off, KS)], sem_o[b])

        @pl.when(i + 2 < NKS)
        def _():
            ip2 = i + 2
            pltpu.async_copy(h.at[isv_all.at[ip2]], hs[b], sem_a[b])
            pltpu.async_copy(h.at[idv_all.at[ip2]], hd[b], sem_b[b])

    def pair_body(j, carry):
        half(2 * j, 0)
        half(2 * j + 1, 1)
        return carry

    lax.fori_loop(0, NKS // 2, pair_body, 0)

    pltpu.make_async_copy(p[0], out_p.at[pl.ds(0, KS)], so0).wait()
    pltpu.make_async_copy(p[1], out_p.at[pl.ds(0, KS)], so1).wait()

    ca = pltpu.async_copy(h.at[isv_t], hs_t, sa0)
    cb = pltpu.async_copy(h.at[idv_t], hd_t, sb0)
    ca.wait()
    cb.wait()
    compute(hs_t, hd_t, p_t, TS)
    pltpu.sync_copy(p_t, out_p.at[pl.ds(base + NKS * KS, TS)])


BN = 632


def _dense_body(x_ref, a0_ref, a1_ref, df_ref, ws_ref, wn_ref, b_ref, o_ref):
    a = a0_ref[0] + a1_ref[0]
    hn = a / df_ref[...]
    h = jnp.dot(x_ref[...], ws_ref[...], preferred_element_type=jnp.float32)
    h = h + jnp.dot(hn, wn_ref[...], preferred_element_type=jnp.float32)
    o_ref[...] = h + b_ref[...]


def _dense(x_p, agg_p, deg_full, ws_t, wn_t, b):
    return pl.pallas_call(
        _dense_body,
        grid=(NPAD // BN,),
        in_specs=[
            pl.BlockSpec((BN, D), lambda i: (i, 0)),
            pl.BlockSpec((1, BN, D), lambda i: (0, i, 0)),
            pl.BlockSpec((1, BN, D), lambda i: (1, i, 0)),
            pl.BlockSpec((BN, D), lambda i: (i, 0)),
            pl.BlockSpec((D, D), lambda i: (0, 0)),
            pl.BlockSpec((D, D), lambda i: (0, 0)),
            pl.BlockSpec((1, D), lambda i: (0, 0)),
        ],
        out_specs=pl.BlockSpec((BN, D), lambda i: (i, 0)),
        out_shape=jax.ShapeDtypeStruct((NPAD, D), jnp.float32),
    )(x_p, agg_p, agg_p, deg_full, ws_t, wn_t, b)


PROWS = E * 16 // D
BR = 4000


def _finish_body(p_ref, g_ref, o_ref, mn_ref, mx_ref):
    ph = pl.program_id(0)
    i = pl.program_id(1)
    s = jnp.dot(p_ref[...], g_ref[...], preferred_element_type=jnp.float32)

    @pl.when(ph == 0)
    def _():
        m = jnp.min(s)
        mm = jnp.max(s)

        @pl.when(i == 0)
        def _():
            mn_ref[0, 0] = m
            mx_ref[0, 0] = mm

        @pl.when(i > 0)
        def _():
            mn_ref[0, 0] = jnp.minimum(mn_ref[0, 0], m)
            mx_ref[0, 0] = jnp.maximum(mx_ref[0, 0], mm)

    @pl.when(ph == 1)
    def _():
        o_ref[...] = (s - mn_ref[0, 0]) / (mx_ref[0, 0] - mn_ref[0, 0])


def _finish(p2, g):
    return pl.pallas_call(
        _finish_body,
        grid=(2, PROWS // BR),
        in_specs=[
            pl.BlockSpec((BR, D), lambda p, i: (i, 0)),
            pl.BlockSpec((D, D), lambda p, i: (0, 0)),
        ],
        out_specs=pl.BlockSpec((BR, D), lambda p, i: (i, 0)),
        out_shape=jax.ShapeDtypeStruct((PROWS, D), jnp.float32),
        scratch_shapes=[
            pltpu.SMEM((1, 1), jnp.float32),
            pltpu.SMEM((1, 1), jnp.float32),
        ],
    )(p2, g)


@jax.jit
def kernel(x, edge_index, W1_self, W1_neigh, b1, W2_self, W2_neigh, b2):
    src = edge_index[0]
    dst = edge_index[1]
    e3 = edge_index.reshape(2, NW, EPT)
    main_s = e3[:, :, : NKS * KS].reshape(2, NW, NKS, KS)
    tail = e3[:, :, NKS * KS:]

    x_p = jnp.pad(x, ((0, NPAD - N), (0, 0)))
    z2d = jnp.zeros((NPAD, D), jnp.float32)
    z1d = jnp.zeros((NPAD,), jnp.float32)

    agg1, deg = _sc_aggregate(x_p, src, dst, z2d, z1d)
    degsum = jnp.maximum(deg[0] + deg[1], 1.0)
    deg_full = jnp.broadcast_to(degsum[:, None], (NPAD, D))

    h1 = _dense(x_p, agg1, deg_full, W1_self.T, W1_neigh.T, b1[None, :])
    agg2, _ = _sc_aggregate(h1, src, dst, z2d, z1d)
    h2 = _dense(h1, agg2, deg_full, W2_self.T, W2_neigh.T, b2[None, :])

    p = _sc_score(h2, main_s[0], main_s[1], tail[0], tail[1])
    p2 = p.reshape(PROWS, D)
    col = jnp.arange(D, dtype=jnp.int32)
    g = (col[:, None] // 16 == col[None, :] % 8).astype(jnp.float32)
    label = _finish(p2, g)[:, :8].reshape(E)
    return label


# speedup vs baseline: 5.3720x; 1.0115x over previous
import functools

import jax
import jax.numpy as jnp
from jax import lax
from jax.experimental import pallas as pl
from jax.experimental.pallas import tpu as pltpu
from jax.experimental.pallas import tpu_sc as plsc

N = 10000
E = 320000
D = 128

NC = 2
NS = 16
NW = NC * NS

NPAD = 10112
STRIPE = NPAD // NS

EPT = E // NW
KA = 64
NKA = EPT // KA
TA = EPT - NKA * KA

KS = 32
NKS = EPT // KS
TS = EPT - NKS * KS

_mesh = plsc.VectorSubcoreMesh(
    core_axis_name="c", subcore_axis_name="s", num_cores=NC, num_subcores=NS
)


@functools.partial(
    pl.kernel,
    out_type=(
        jax.ShapeDtypeStruct((NC, NPAD, D), jnp.float32),
        jax.ShapeDtypeStruct((NC, NPAD), jnp.float32),
    ),
    mesh=_mesh,
    scratch_types=[
        pltpu.VMEM_SHARED((NPAD, D), jnp.float32),
        pltpu.VMEM_SHARED((NPAD,), jnp.float32),
        pltpu.VMEM((KA, D), jnp.float32),
        pltpu.VMEM((KA, D), jnp.float32),
        pltpu.VMEM((KA, D), jnp.float32),
        pltpu.VMEM((KA,), jnp.int32),
        pltpu.VMEM((KA,), jnp.int32),
        pltpu.VMEM((KA,), jnp.int32),
        pltpu.VMEM((KA,), jnp.int32),
        pltpu.VMEM((KA,), jnp.int32),
        pltpu.VMEM((KA,), jnp.int32),
        pltpu.VMEM((KA,), jnp.float32),
        pltpu.VMEM((TA, D), jnp.float32),
        pltpu.VMEM((TA,), jnp.int32),
        pltpu.VMEM((TA,), jnp.int32),
        pltpu.VMEM((TA,), jnp.float32),
        pltpu.SemaphoreType.DMA,
        pltpu.SemaphoreType.DMA,
        pltpu.SemaphoreType.DMA,
        pltpu.SemaphoreType.DMA,
        pltpu.SemaphoreType.DMA,
        pltpu.SemaphoreType.DMA,
        pltpu.SemaphoreType.DMA,
        pltpu.SemaphoreType.DMA,
        pltpu.SemaphoreType.DMA,
    ],
)
def _sc_aggregate(feat, srcl, dstl, z2d, z1d,
                  out_agg, out_deg,
                  agg_sh, deg_sh, rows0, rows1, rows2,
                  is0, is1, is2, id0, id1, id2, ones_v,
                  rows_t, isv_t, idv_t, ones_t,
                  si0, si1, si2, sg0, sg1, sg2, ss0, ss1, ss2):
    c = lax.axis_index("c")
    s = lax.axis_index("s")
    wid = c * NS + s
    rows = [rows0, rows1, rows2]
    isv = [is0, is1, is2]
    idv = [id0, id1, id2]
    sem_i = [si0, si1, si2]
    sem_g = [sg0, sg1, sg2]
    sem_s = [ss0, ss1, ss2]

    pltpu.sync_copy(z2d.at[pl.ds(s * STRIPE, STRIPE)],
                    agg_sh.at[pl.ds(s * STRIPE, STRIPE)])

    @pl.when(s == 0)
    def _():
        pltpu.sync_copy(z1d, deg_sh)

    for i in range(KA // 16):
        ones_v[pl.ds(i * 16, 16)] = jnp.full((16,), 1.0, jnp.float32)
    ones_t[...] = jnp.full((TA,), 1.0, jnp.float32)

    plsc.subcore_barrier()

    base = wid * EPT

    desc_i = [None] * NKA
    desc_g = [None] * NKA
    desc_s = [None] * NKA

    for t in range(NKA + 2):
        b = t % 3
        if t >= 3:
            desc_s[t - 3][0].wait()
            desc_s[t - 3][1].wait()
        if t < NKA:
            off = pl.multiple_of(base + t * KA, 8)
            d1 = pltpu.async_copy(srcl.at[pl.ds(off, KA)], isv[b], sem_i[b])
            d2 = pltpu.async_copy(dstl.at[pl.ds(off, KA)], idv[b], sem_i[b])
            desc_i[t] = (d1, d2)
        if 0 <= t - 1 < NKA:
            g = t - 1
            bg = g % 3
            desc_i[g][0].wait()
            desc_i[g][1].wait()
            desc_g[g] = pltpu.async_copy(feat.at[isv[bg]], rows[bg], sem_g[bg])
        if 0 <= t - 2 < NKA:
            sc = t - 2
            bs = sc % 3
            desc_g[sc].wait()
            d1 = pltpu.async_copy(rows[bs], agg_sh.at[idv[bs]], sem_s[bs],
                                  add=True)
            d2 = pltpu.async_copy(ones_v, deg_sh.at[idv[bs]], sem_s[bs],
                                  add=True)
            desc_s[sc] = (d1, d2)
    desc_s[NKA - 1][0].wait()
    desc_s[NKA - 1][1].wait()

    toff = pl.multiple_of(base + NKA * KA, 8)
    pltpu.sync_copy(srcl.at[pl.ds(toff, TA)], isv_t)
    pltpu.sync_copy(dstl.at[pl.ds(toff, TA)], idv_t)
    pltpu.async_copy(feat.at[isv_t], rows_t, sg0).wait()
    pltpu.sync_copy(rows_t, agg_sh.at[idv_t], add=True)
    pltpu.sync_copy(ones_t, deg_sh.at[idv_t], add=True)

    plsc.subcore_barrier()

    pltpu.sync_copy(agg_sh.at[pl.ds(s * STRIPE, STRIPE)],
                    out_agg.at[c, pl.ds(s * STRIPE, STRIPE)])

    @pl.when(s == 0)
    def _():
        pltpu.sync_copy(deg_sh, out_deg.at[c])


@functools.partial(
    pl.kernel,
    out_type=jax.ShapeDtypeStruct((E, 16), jnp.float32),
    mesh=_mesh,
    scratch_types=[
        pltpu.VMEM((NKS // 4, 4 * KS), jnp.int32),
        pltpu.VMEM((NKS // 4, 4 * KS), jnp.int32),
        pltpu.VMEM((KS, D), jnp.float32),
        pltpu.VMEM((KS, D), jnp.float32),
        pltpu.VMEM((KS, D), jnp.float32),
        pltpu.VMEM((KS, D), jnp.float32),
        pltpu.VMEM((KS, D), jnp.float32),
        pltpu.VMEM((KS, D), jnp.float32),
        pltpu.VMEM((KS, D), jnp.float32),
        pltpu.VMEM((KS, D), jnp.float32),
        pltpu.VMEM((KS, 16), jnp.float32),
        pltpu.VMEM((KS, 16), jnp.float32),
        pltpu.VMEM((KS, 16), jnp.float32),
        pltpu.VMEM((KS, 16), jnp.float32),
        pltpu.VMEM((TS, D), jnp.float32),
        pltpu.VMEM((TS, D), jnp.float32),
        pltpu.VMEM((TS,), jnp.int32),
        pltpu.VMEM((TS,), jnp.int32),
        pltpu.VMEM((TS, 16), jnp.float32),
        pltpu.SemaphoreType.DMA,
        pltpu.SemaphoreType.DMA,
        pltpu.SemaphoreType.DMA,
        pltpu.SemaphoreType.DMA,
        pltpu.SemaphoreType.DMA,
        pltpu.SemaphoreType.DMA,
        pltpu.SemaphoreType.DMA,
        pltpu.SemaphoreType.DMA,
        pltpu.SemaphoreType.DMA,
        pltpu.SemaphoreType.DMA,
        pltpu.SemaphoreType.DMA,
        pltpu.SemaphoreType.DMA,
    ],
)
def _sc_score(h, src_s, dst_s, src_st, dst_st, out_p,
              isv_all, idv_all, hs0, hs1, hs2, hs3, hd0, hd1, hd2, hd3,
              p0, p1, p2, p3,
              hs_t, hd_t, isv_t, idv_t, p_t,
              sa0, sa1, sa2, sa3, sb0, sb1, sb2, sb3, so0, so1, so2, so3):
    c = lax.axis_index("c")
    s = lax.axis_index("s")
    wid = c * NS + s
    base = wid * EPT
    hs = [hs0, hs1, hs2, hs3]
    hd = [hd0, hd1, hd2, hd3]
    p = [p0, p1, p2, p3]
    sem_a = [sa0, sa1, sa2, sa3]
    sem_b = [sb0, sb1, sb2, sb3]
    sem_o = [so0, so1, so2, so3]
    NB = 4

    pltpu.sync_copy(src_s.at[wid], isv_all)
    pltpu.sync_copy(dst_s.at[wid], idv_all)
    pltpu.sync_copy(src_st.at[wid], isv_t)
    pltpu.sync_copy(dst_st.at[wid], idv_t)

    def compute(hsr, hdr, pr, k):
        for e in range(k):
            acc = hsr[e, pl.ds(0, 16)] * hdr[e, pl.ds(0, 16)]
            for j in range(1, D // 16):
                acc = acc + hsr[e, pl.ds(j * 16, 16)] * hdr[e, pl.ds(j * 16, 16)]
            pr[e, :] = acc

    NGRP = NKS // NB

    for b in range(NB):
        pltpu.async_copy(h.at[isv_all.at[0, pl.ds(b * KS, KS)]],
                         hs[b], sem_a[b])
        pltpu.async_copy(h.at[idv_all.at[0, pl.ds(b * KS, KS)]],
                         hd[b], sem_b[b])

    def step(j, b):
        i = NB * j + b
        @pl.when(j >= 1)
        def _():
            pltpu.make_async_copy(
                p[b], out_p.at[pl.ds(0, KS)], sem_o[b]).wait()

        pltpu.make_async_copy(h.at[pl.ds(0, KS)], hs[b], sem_a[b]).wait()
        pltpu.make_async_copy(h.at[pl.ds(0, KS)], hd[b], sem_b[b]).wait()

        compute(hs[b], hd[b], p[b], KS)

        off = pl.multiple_of(base + i * KS, 8)
        pltpu.async_copy(p[b], out_p.at[pl.ds(off, KS)], sem_o[b])

        @pl.when(j + 1 < NGRP)
        def _():
            jp1 = j + 1
            pltpu.async_copy(h.at[isv_all.at[jp1, pl.ds(b * KS, KS)]],
                             hs[b], sem_a[b])
            pltpu.async_copy(h.at[idv_all.at[jp1, pl.ds(b * KS, KS)]],
                             hd[b], sem_b[b])

    def group_body(j, carry):
        for b in range(NB):
            step(j, b)
        return carry

    lax.fori_loop(0, NGRP, group_body, 0)

    for b in range(NB):
        pltpu.make_async_copy(p[b], out_p.at[pl.ds(0, KS)], sem_o[b]).wait()

    ca = pltpu.async_copy(h.at[isv_t], hs_t, sa0)
    cb = pltpu.async_copy(h.at[idv_t], hd_t, sb0)
    ca.wait()
    cb.wait()
    compute(hs_t, hd_t, p_t, TS)
    pltpu.sync_copy(p_t, out_p.at[pl.ds(base + NKS * KS, TS)])


BN = 1000


def _dense_body(x_ref, a0_ref, a1_ref, dg_ref, ws_ref, wn_ref, b_ref, o_ref):
    a = a0_ref[0] + a1_ref[0]
    hn = a * (1.0 / dg_ref[...])
    h = jnp.dot(x_ref[...], ws_ref[...], preferred_element_type=jnp.float32)
    h = h + jnp.dot(hn, wn_ref[...], preferred_element_type=jnp.float32)
    o_ref[...] = h + b_ref[...]


def _dense(x, agg_p, degsum2, ws_t, wn_t, b):
    return pl.pallas_call(
        _dense_body,
        grid=(N // BN,),
        in_specs=[
            pl.BlockSpec((BN, D), lambda i: (i, 0)),
            pl.BlockSpec((1, BN, D), lambda i: (0, i, 0)),
            pl.BlockSpec((1, BN, D), lambda i: (1, i, 0)),
            pl.BlockSpec((BN, 1), lambda i: (i, 0)),
            pl.BlockSpec((D, D), lambda i: (0, 0)),
            pl.BlockSpec((D, D), lambda i: (0, 0)),
            pl.BlockSpec((1, D), lambda i: (0, 0)),
        ],
        out_specs=pl.BlockSpec((BN, D), lambda i: (i, 0)),
        out_shape=jax.ShapeDtypeStruct((N, D), jnp.float32),
    )(x, agg_p, agg_p, degsum2, ws_t, wn_t, b)


PROWS = E * 16 // D
BR = 4000


def _finish_body(p_ref, g_ref, o_ref, mn_ref, mx_ref):
    ph = pl.program_id(0)
    i = pl.program_id(1)
    s = jnp.dot(p_ref[...], g_ref[...], preferred_element_type=jnp.float32)

    @pl.when(ph == 0)
    def _():
        m = jnp.min(s)
        mm = jnp.max(s)

        @pl.when(i == 0)
        def _():
            mn_ref[0, 0] = m
            mx_ref[0, 0] = mm

        @pl.when(i > 0)
        def _():
            mn_ref[0, 0] = jnp.minimum(mn_ref[0, 0], m)
            mx_ref[0, 0] = jnp.maximum(mx_ref[0, 0], mm)

    @pl.when(ph == 1)
    def _():
        scale = 1.0 / (mx_ref[0, 0] - mn_ref[0, 0])
        o_ref[...] = lax.slice((s - mn_ref[0, 0]) * scale, (0, 0), (BR, 8))


def _finish(p2, g):
    return pl.pallas_call(
        _finish_body,
        grid=(2, PROWS // BR),
        in_specs=[
            pl.BlockSpec((BR, D), lambda p, i: (i, 0)),
            pl.BlockSpec((D, D), lambda p, i: (0, 0)),
        ],
        out_specs=pl.BlockSpec((BR, 8), lambda p, i: (i, 0)),
        out_shape=jax.ShapeDtypeStruct((PROWS, 8), jnp.float32),
        scratch_shapes=[
            pltpu.SMEM((1, 1), jnp.float32),
            pltpu.SMEM((1, 1), jnp.float32),
        ],
    )(p2, g)


@jax.jit
def kernel(x, edge_index, W1_self, W1_neigh, b1, W2_self, W2_neigh, b2):
    src = edge_index[0]
    dst = edge_index[1]
    e3 = edge_index.reshape(2, NW, EPT)
    main_s = e3[:, :, : NKS * KS].reshape(2, NW, NKS // 4, 4 * KS)
    tail = e3[:, :, NKS * KS:]

    z2d = jnp.zeros((NPAD, D), jnp.float32)
    z1d = jnp.zeros((NPAD,), jnp.float32)

    agg1, deg = _sc_aggregate(x, src, dst, z2d, z1d)
    degsum2 = jnp.maximum(deg[0] + deg[1], 1.0)[:N, None]

    h1 = _dense(x, agg1, degsum2, W1_self.T, W1_neigh.T, b1[None, :])
    agg2, _ = _sc_aggregate(h1, src, dst, z2d, z1d)
    h2 = _dense(h1, agg2, degsum2, W2_self.T, W2_neigh.T, b2[None, :])

    p = _sc_score(h2, main_s[0], main_s[1], tail[0], tail[1])
    p2 = p.reshape(PROWS, D)
    col = jnp.arange(D, dtype=jnp.int32)
    g = (col[:, None] // 16 == col[None, :] % 8).astype(jnp.float32)
    label = _finish(p2, g).reshape(E)
    return label
